# baseline (speedup 1.0000x reference)
import functools
import numpy as np
import jax
import jax.numpy as jnp
from jax import lax
from jax.experimental import pallas as pl
from jax.experimental.pallas import tpu as pltpu
from jax.experimental.pallas import tpu_sc as plsc

_STEPS = 1000
_N = 128
_NW = 32
_SPW = _N // _NW
_ROWS = 56
_CHUNKS_PER_CH = 224 // _ROWS


def _schedule_tables():
    scale = np.float32(1000.0 / _STEPS)
    beta = np.linspace(scale * np.float32(0.0001), scale * np.float32(0.02),
                       _STEPS, dtype=np.float32)
    alpha = (np.float32(1.0) - beta).astype(np.float32)
    alpha_cum = np.cumprod(alpha, dtype=np.float32)
    sqrt_ac = np.sqrt(alpha_cum).astype(np.float32)
    sqrt_omac = np.sqrt((np.float32(1.0) - alpha_cum)).astype(np.float32)
    return sqrt_ac, sqrt_omac


_SQRT_AC, _SQRT_OMAC = _schedule_tables()
_SA16 = np.ascontiguousarray(np.repeat(_SQRT_AC[:, None], 128, axis=1))
_SO16 = np.ascontiguousarray(np.repeat(_SQRT_OMAC[:, None], 128, axis=1))

_kernel_cache = {}


def _get_sc_kernel(shape):
    if shape in _kernel_cache:
        return _kernel_cache[shape]
    n, c, h, w = shape
    mesh = plsc.VectorSubcoreMesh(core_axis_name="c", subcore_axis_name="s")

    @functools.partial(
        pl.kernel,
        mesh=mesh,
        out_type=jax.ShapeDtypeStruct(shape, jnp.float32),
        scratch_types=[
            pltpu.VMEM((n,), jnp.int32),
            pltpu.VMEM((n, 128), jnp.float32),
            pltpu.VMEM((n, 128), jnp.float32),
            pltpu.VMEM((_ROWS, w), jnp.float32),
            pltpu.VMEM((_ROWS, w), jnp.float32),
            pltpu.VMEM((_ROWS, w), jnp.float32),
            pltpu.SemaphoreType.DMA,
            pltpu.SemaphoreType.DMA,
        ],
    )
    def _sc_diffusion(x_hbm, e_hbm, ts_hbm, sa_hbm, so_hbm, o_hbm,
                      idx_v, af_v, cf_v, xb, ebuf, ob, s1, s2):
        wid = lax.axis_index("s") * 2 + lax.axis_index("c")

        pltpu.sync_copy(ts_hbm, idx_v)
        pltpu.async_copy(sa_hbm.at[idx_v], af_v, s1).wait()
        pltpu.async_copy(so_hbm.at[idx_v], cf_v, s2).wait()

        def per_sample(k, _):
            gi = wid * _SPW + k
            a_vec = af_v[gi, pl.ds(0, 16)]
            c_vec = cf_v[gi, pl.ds(0, 16)]

            def per_chunk(q, _):
                ch = q // _CHUNKS_PER_CH
                r0 = (q % _CHUNKS_PER_CH) * _ROWS
                pltpu.sync_copy(x_hbm.at[gi, ch, pl.ds(r0, _ROWS)], xb)
                pltpu.sync_copy(e_hbm.at[gi, ch, pl.ds(r0, _ROWS)], ebuf)

                def per_row(r, _):
                    for cc in range(w // 16):
                        sl = pl.ds(cc * 16, 16)
                        ob[r, sl] = a_vec * xb[r, sl] + c_vec * ebuf[r, sl]
                    return 0

                lax.fori_loop(0, _ROWS, per_row, 0)
                pltpu.sync_copy(ob, o_hbm.at[gi, ch, pl.ds(r0, _ROWS)])
                return 0

            lax.fori_loop(0, c * _CHUNKS_PER_CH, per_chunk, 0)
            return 0

        lax.fori_loop(0, _SPW, per_sample, 0)

    _kernel_cache[shape] = _sc_diffusion
    return _sc_diffusion


def kernel(inputs, eps, ts):
    sa = jnp.asarray(_SA16)
    so = jnp.asarray(_SO16)
    return _get_sc_kernel(inputs.shape)(inputs, eps, ts, sa, so)


# speedup vs baseline: 1.0657x; 1.0657x over previous
import functools
import numpy as np
import jax
import jax.numpy as jnp
from jax import lax
from jax.experimental import pallas as pl
from jax.experimental.pallas import tpu as pltpu
from jax.experimental.pallas import tpu_sc as plsc

_STEPS = 1000
_N = 128
_NW = 32
_SPW = _N // _NW
_ROWS = 56
_CPC = 224 // _ROWS


def _schedule_tables():
    scale = np.float32(1000.0 / _STEPS)
    beta = np.linspace(scale * np.float32(0.0001), scale * np.float32(0.02),
                       _STEPS, dtype=np.float32)
    alpha = (np.float32(1.0) - beta).astype(np.float32)
    alpha_cum = np.cumprod(alpha, dtype=np.float32)
    sqrt_ac = np.sqrt(alpha_cum).astype(np.float32)
    sqrt_omac = np.sqrt((np.float32(1.0) - alpha_cum)).astype(np.float32)
    return sqrt_ac, sqrt_omac


_SQRT_AC, _SQRT_OMAC = _schedule_tables()
_SA128 = np.ascontiguousarray(np.repeat(_SQRT_AC[:, None], 128, axis=1))
_SO128 = np.ascontiguousarray(np.repeat(_SQRT_OMAC[:, None], 128, axis=1))

_kernel_cache = {}


def _get_sc_kernel(shape):
    if shape in _kernel_cache:
        return _kernel_cache[shape]
    n, c, h, w = shape
    nchunks = _SPW * c * _CPC
    mesh = plsc.VectorSubcoreMesh(core_axis_name="c", subcore_axis_name="s")

    @functools.partial(
        pl.kernel,
        mesh=mesh,
        out_type=jax.ShapeDtypeStruct(shape, jnp.float32),
        scratch_types=[
            pltpu.VMEM((8,), jnp.int32),
            pltpu.VMEM((8, 128), jnp.float32),
            pltpu.VMEM((8, 128), jnp.float32),
            pltpu.VMEM((2, _ROWS, w), jnp.float32),
            pltpu.VMEM((2, _ROWS, w), jnp.float32),
            pltpu.VMEM((2, _ROWS, w), jnp.float32),
            pltpu.SemaphoreType.DMA,
            pltpu.SemaphoreType.DMA,
            pltpu.SemaphoreType.DMA,
            pltpu.SemaphoreType.DMA,
            pltpu.SemaphoreType.DMA,
            pltpu.SemaphoreType.DMA,
        ],
    )
    def _sc_diffusion(x_hbm, e_hbm, ts_hbm, sa_hbm, so_hbm, o_hbm,
                      idx_v, af_v, cf_v, xb, ebuf, ob,
                      sx0, sx1, se0, se1, so0, so1):
        wid = lax.axis_index("s") * 2 + lax.axis_index("c")
        sxs = (sx0, sx1)
        ses = (se0, se1)
        sos = (so0, so1)

        pltpu.sync_copy(ts_hbm.at[pl.ds((wid // 2) * 8, 8)], idx_v)
        pltpu.async_copy(sa_hbm.at[idx_v], af_v, sx0).wait()
        pltpu.async_copy(so_hbm.at[idx_v], cf_v, sx1).wait()

        def decode(t):
            k = t // (c * _CPC)
            q = t % (c * _CPC)
            ch = q // _CPC
            r0 = (q % _CPC) * _ROWS
            gi = wid * _SPW + k
            return gi, ch, r0, (wid % 2) * _SPW + k

        def start_in(t, slot):
            gi, ch, r0, _ = decode(t)
            src = (gi, ch, pl.ds(r0, _ROWS))
            pltpu.make_async_copy(x_hbm.at[src], xb.at[slot], sxs[slot]).start()
            pltpu.make_async_copy(e_hbm.at[src], ebuf.at[slot], ses[slot]).start()

        start_in(0, 0)
        start_in(1, 1)

        def do_chunk(t, slot):
            gi, ch, r0, ri = decode(t)
            src = (gi, ch, pl.ds(r0, _ROWS))
            a_vec = af_v[ri, pl.ds(0, 16)]
            c_vec = cf_v[ri, pl.ds(0, 16)]

            @pl.when(t >= 2)
            def _():
                pgi, pch, pr0, _ = decode(t - 2)
                pltpu.make_async_copy(
                    ob.at[slot], o_hbm.at[pgi, pch, pl.ds(pr0, _ROWS)],
                    sos[slot]).wait()

            pltpu.make_async_copy(x_hbm.at[src], xb.at[slot],
                                  sxs[slot]).wait()
            pltpu.make_async_copy(e_hbm.at[src], ebuf.at[slot],
                                  ses[slot]).wait()

            def per_row(r, _):
                for cc in range(w // 16):
                    sl = pl.ds(cc * 16, 16)
                    ob[slot, r, sl] = (a_vec * xb[slot, r, sl]
                                       + c_vec * ebuf[slot, r, sl])
                return 0

            lax.fori_loop(0, _ROWS, per_row, 0)
            pltpu.make_async_copy(ob.at[slot], o_hbm.at[src], sos[slot]).start()

            @pl.when(t + 2 < nchunks)
            def _():
                start_in(t + 2, slot)

        def pair(g, _):
            do_chunk(2 * g, 0)
            do_chunk(2 * g + 1, 1)
            return 0

        lax.fori_loop(0, nchunks // 2, pair, 0)

        for t, slot in ((nchunks - 2, 0), (nchunks - 1, 1)):
            gi, ch, r0, _ = decode(t)
            pltpu.make_async_copy(
                ob.at[slot], o_hbm.at[gi, ch, pl.ds(r0, _ROWS)],
                sos[slot]).wait()

    _kernel_cache[shape] = _sc_diffusion
    return _sc_diffusion


def kernel(inputs, eps, ts):
    sa = jnp.asarray(_SA128)
    so = jnp.asarray(_SO128)
    return _get_sc_kernel(inputs.shape)(inputs, eps, ts, sa, so)


# speedup vs baseline: 1.3495x; 1.2663x over previous
import functools
import numpy as np
import jax
import jax.numpy as jnp
from jax import lax
from jax.experimental import pallas as pl
from jax.experimental.pallas import tpu as pltpu
from jax.experimental.pallas import tpu_sc as plsc

_STEPS = 1000
_BB = 8


def _schedule_tables():
    scale = np.float32(1000.0 / _STEPS)
    beta = np.linspace(scale * np.float32(0.0001), scale * np.float32(0.02),
                       _STEPS, dtype=np.float32)
    alpha = (np.float32(1.0) - beta).astype(np.float32)
    alpha_cum = np.cumprod(alpha, dtype=np.float32)
    sqrt_ac = np.sqrt(alpha_cum).astype(np.float32)
    sqrt_omac = np.sqrt((np.float32(1.0) - alpha_cum)).astype(np.float32)
    return sqrt_ac, sqrt_omac


_SQRT_AC, _SQRT_OMAC = _schedule_tables()
_SA128 = np.ascontiguousarray(np.repeat(_SQRT_AC[:, None], 128, axis=1))
_SO128 = np.ascontiguousarray(np.repeat(_SQRT_OMAC[:, None], 128, axis=1))

_sc_cache = {}


def _get_sc_gather(n):
    if n in _sc_cache:
        return _sc_cache[n]
    mesh = plsc.VectorSubcoreMesh(core_axis_name="c", subcore_axis_name="s")

    @functools.partial(
        pl.kernel,
        mesh=mesh,
        out_type=[
            jax.ShapeDtypeStruct((n, 128), jnp.float32),
            jax.ShapeDtypeStruct((n, 128), jnp.float32),
        ],
        scratch_types=[
            pltpu.VMEM((8,), jnp.int32),
            pltpu.VMEM((8, 128), jnp.float32),
            pltpu.VMEM((8, 128), jnp.float32),
            pltpu.SemaphoreType.DMA,
            pltpu.SemaphoreType.DMA,
        ],
    )
    def _sc_gather(ts_hbm, sa_hbm, so_hbm, a_out, c_out,
                   idx_v, a_v, c_v, s1, s2):
        wid = lax.axis_index("s") * 2 + lax.axis_index("c")

        @pl.when(wid < n // 8)
        def _():
            base = wid * 8
            pltpu.sync_copy(ts_hbm.at[pl.ds(base, 8)], idx_v)
            pltpu.async_copy(sa_hbm.at[idx_v], a_v, s1).wait()
            pltpu.async_copy(so_hbm.at[idx_v], c_v, s2).wait()
            pltpu.sync_copy(a_v, a_out.at[pl.ds(base, 8)])
            pltpu.sync_copy(c_v, c_out.at[pl.ds(base, 8)])

    _sc_cache[n] = _sc_gather
    return _sc_gather


def _scale_add_kernel(a_ref, c_ref, x_ref, e_ref, o_ref):
    g = pl.program_id(0)
    for j in range(_BB):
        i = g * _BB + j
        o_ref[j] = a_ref[i] * x_ref[j] + c_ref[i] * e_ref[j]


def kernel(inputs, eps, ts):
    n, c, h, w = inputs.shape

    sa = jnp.asarray(_SA128)
    so = jnp.asarray(_SO128)
    a_rows, c_rows = _get_sc_gather(n)(ts, sa, so)
    coef_a = a_rows[:, 0]
    coef_c = c_rows[:, 0]

    blk = (_BB, c, h, w)
    out = pl.pallas_call(
        _scale_add_kernel,
        grid_spec=pltpu.PrefetchScalarGridSpec(
            num_scalar_prefetch=2,
            grid=(n // _BB,),
            in_specs=[
                pl.BlockSpec(blk, lambda b, *_: (b, 0, 0, 0)),
                pl.BlockSpec(blk, lambda b, *_: (b, 0, 0, 0)),
            ],
            out_specs=pl.BlockSpec(blk, lambda b, *_: (b, 0, 0, 0)),
        ),
        out_shape=jax.ShapeDtypeStruct(inputs.shape, jnp.float32),
    )(coef_a, coef_c, inputs, eps)
    return out
